# Initial kernel scaffold; baseline (speedup 1.0000x reference)
import jax
import jax.numpy as jnp
from jax.experimental import pallas as pl


def kernel(x, A, mask, weight, ct_v, ct_g, ct_b, bias):
    raise NotImplementedError("write your pallas kernel here")


# speedup vs baseline: 2.2238x; 2.2238x over previous
import jax
import jax.numpy as jnp
from jax import lax
from jax.experimental import pallas as pl
from jax.experimental.pallas import tpu as pltpu

_INC = 1024
_INN = 512
_OUTC = 2048
_OUTN = 64
_D = 4
_N = 32


def _fgl_body(x_ref, at_ref, maskt_ref, wt_ref, ctv_ref, ctg_ref, ctb_ref,
              bias_ref, y_ref, s_ref, wn_ref):
    n = pl.program_id(0)

    @pl.when(n == 0)
    def _init():
        iota = lax.broadcasted_iota(jnp.int32, (_INN, _OUTN), 0)
        s = jnp.zeros((_INN, _OUTN), jnp.float32)
        for d in range(_D):
            a_row = at_ref[d, :][None, :]
            m_row = maskt_ref[d, :][None, :]
            s = s + jnp.where(iota == a_row, m_row, 0.0)
        s_ref[...] = s
        v = ctv_ref[...]
        rn = jnp.sqrt(jnp.sum(v * v, axis=1, keepdims=True))
        wn_ref[...] = v * (ctg_ref[...] / rn)

    xn = x_ref[0]
    t = jnp.dot(xn, s_ref[...], preferred_element_type=jnp.float32)
    scaled = t * wt_ref[...]
    out = jnp.dot(wn_ref[...], scaled, preferred_element_type=jnp.float32)
    y_ref[0] = out + ctb_ref[...] + bias_ref[...]


def kernel(x, A, mask, weight, ct_v, ct_g, ct_b, bias):
    at = jnp.full((8, _OUTN), -1, dtype=jnp.int32).at[:_D].set(A.T.astype(jnp.int32))
    maskt = jnp.zeros((8, _OUTN), jnp.float32).at[:_D].set(mask[:, :, 0].T)
    wt = weight[:, 0, :].T
    ctg = ct_g[:, None]
    ctb = ct_b[:, None]

    y = pl.pallas_call(
        _fgl_body,
        grid=(_N,),
        in_specs=[
            pl.BlockSpec((1, _INC, _INN), lambda n: (n, 0, 0)),
            pl.BlockSpec((8, _OUTN), lambda n: (0, 0)),
            pl.BlockSpec((8, _OUTN), lambda n: (0, 0)),
            pl.BlockSpec((_INC, _OUTN), lambda n: (0, 0)),
            pl.BlockSpec((_OUTC, _INC), lambda n: (0, 0)),
            pl.BlockSpec((_OUTC, 1), lambda n: (0, 0)),
            pl.BlockSpec((_OUTC, 1), lambda n: (0, 0)),
            pl.BlockSpec((_OUTC, _OUTN), lambda n: (0, 0)),
        ],
        out_specs=pl.BlockSpec((1, _OUTC, _OUTN), lambda n: (n, 0, 0)),
        out_shape=jax.ShapeDtypeStruct((_N, _OUTC, _OUTN), jnp.float32),
        scratch_shapes=[
            pltpu.VMEM((_INN, _OUTN), jnp.float32),
            pltpu.VMEM((_OUTC, _INC), jnp.float32),
        ],
    )(x, at, maskt, wt, ct_v, ctg, ctb, bias)
    return y
